# baseline (speedup 1.0000x reference)
import functools

import jax
import jax.numpy as jnp
from jax import lax
from jax.experimental import pallas as pl
from jax.experimental.pallas import tpu as pltpu
from jax.experimental.pallas import tpu_sc as plsc

_N = 10000
_NP = 10240
_E = 320000
_C = 128
_H = _C // 2
_NC = 2
_NS = 16
_NW = _NC * _NS
_L = 16

_EPW = 10240
_EPAD = _EPW * _NW
_EPT = _EPW * _NC
_CH = 2048
_KB = 128
_NB = _EPW // _KB
_SS = _EPW // _L


_BLK = 1024


def _dot(a, b):
    return jnp.dot(a, b, preferred_element_type=jnp.float32)


def _store_h2(h2_ref, hs_ref, hd_ref, h2, asrc, adst):
    h2_ref[0] = h2[:, 0:_H]
    h2_ref[1] = h2[:, _H:_C]
    i = pl.program_id(0)
    hs_ref[pl.ds(i * _BLK, _BLK)] = jnp.sum(h2 * asrc[None, :], axis=1)
    hd_ref[pl.ds(i * _BLK, _BLK)] = jnp.sum(h2 * adst[None, :], axis=1)


def _combine(o_ref, b_ref):
    return jnp.concatenate(
        [o_ref[0, 0] + o_ref[1, 0], o_ref[0, 1] + o_ref[1, 1]],
        axis=1) + b_ref[...][None, :]


def _prep0_body(x_ref, w1_ref, w2_ref, w_ref, asrc_ref, adst_ref,
                h2_ref, hs_ref, hd_ref):
    t = _dot(_dot(x_ref[...], w1_ref[...]), w2_ref[...])
    h2 = _dot(t, w_ref[...])
    _store_h2(h2_ref, hs_ref, hd_ref, h2, asrc_ref[...], adst_ref[...])


def _prepl_body(o_ref, b_ref, w_ref, asrc_ref, adst_ref,
                h2_ref, hs_ref, hd_ref):
    h2 = _dot(_combine(o_ref, b_ref), w_ref[...])
    _store_h2(h2_ref, hs_ref, hd_ref, h2, asrc_ref[...], adst_ref[...])


def _final_body(o_ref, b_ref, w3_ref, out_ref):
    h = jnp.maximum(_combine(o_ref, b_ref), 0.0)
    out_ref[...] = _dot(h, w3_ref[0:_C, :]) + _dot(h, w3_ref[_C:2 * _C, :])


_mat_spec = pl.BlockSpec((_C, _C), lambda i: (0, 0))
_vec_spec = pl.BlockSpec((_C,), lambda i: (0,))
_row_spec = pl.BlockSpec((_BLK, _C), lambda i: (i, 0))
_h2_spec = pl.BlockSpec((_NC, _BLK, _H), lambda i: (0, i, 0))
_par_spec = pl.BlockSpec((_NC, _NC, _BLK, _H), lambda i: (0, 0, i, 0))
_sca_spec = pl.BlockSpec((_NP,), lambda i: (0,))

_f32 = jnp.float32
_h2_shape = jax.ShapeDtypeStruct((_NC, _NP, _H), _f32)
_nv_shape = jax.ShapeDtypeStruct((_NP,), _f32)

_prep0 = pl.pallas_call(
    _prep0_body,
    grid=(_NP // _BLK,),
    in_specs=[_row_spec, _mat_spec, _mat_spec, _mat_spec, _vec_spec, _vec_spec],
    out_specs=[_h2_spec, _sca_spec, _sca_spec],
    out_shape=[_h2_shape, _nv_shape, _nv_shape],
)

_prepl = pl.pallas_call(
    _prepl_body,
    grid=(_NP // _BLK,),
    in_specs=[_par_spec, _vec_spec, _mat_spec, _vec_spec, _vec_spec],
    out_specs=[_h2_spec, _sca_spec, _sca_spec],
    out_shape=[_h2_shape, _nv_shape, _nv_shape],
)

_final = pl.pallas_call(
    _final_body,
    grid=(_NP // _BLK,),
    in_specs=[_par_spec, _vec_spec,
              pl.BlockSpec((2 * _C, _C), lambda i: (0, 0))],
    out_specs=_row_spec,
    out_shape=jax.ShapeDtypeStruct((_NP, _C), _f32),
)


def _sc_gat_body(h2_hbm, hs_hbm, hd_hbm, src_hbm, dst_hbm,
                 out_hbm,
                 hs_v, hd_v, sa_v, da_v, w_v, srcb_v, dstb_v, id_v, ssum_v,
                 rows_v, rows2_v, s_sh, o_sh, gsem0, gsem1, ssem):
    c = lax.axis_index("c")
    s = lax.axis_index("s")
    wid = s * _NC + c

    def _vgather(x, idx):
        return lax.gather(
            x, idx[:, None],
            lax.GatherDimensionNumbers(offset_dims=(),
                                       collapsed_slice_dims=(0,),
                                       start_index_map=(0,)),
            (1,), mode=lax.GatherScatterMode.PROMISE_IN_BOUNDS)

    zeros16 = jnp.zeros((_L,), _f32)
    iota16 = lax.iota(jnp.int32, _L)

    pltpu.sync_copy(hs_hbm, hs_v)
    pltpu.sync_copy(hd_hbm, hd_v)
    pltpu.sync_copy(src_hbm.at[pl.ds(wid * _EPW, _EPW)], srcb_v)
    for q in range(_NB):
        pltpu.sync_copy(dst_hbm.at[pl.ds(wid * _EPW + q * _KB, _KB)],
                        dstb_v.at[q])

    def _zs(i, _):
        ssum_v[i] = zeros16
        return 0
    lax.fori_loop(0, _SS, _zs, 0)

    pltpu.sync_copy(ssum_v.at[pl.ds(s * (_SS // _NS), _SS // _NS)],
                    s_sh.at[pl.ds(s * (_SS // _NS), _SS // _NS)])

    def _zr(r, _):
        for k in range(_H // _L):
            rows_v[r, pl.ds(k * _L, _L)] = zeros16
        return 0
    lax.fori_loop(0, _KB, _zr, 0)

    def _mx(i, m):
        return jnp.maximum(m, hs_v[pl.ds(i * _L, _L)])
    m16 = lax.fori_loop(0, _NP // _L, _mx, jnp.full((_L,), -3e38, _f32))
    for k in (8, 4, 2, 1):
        m16 = jnp.maximum(m16, _vgather(m16, iota16 ^ k))
    big_m = m16

    def _edge_w(base, i):
        off = i * _L
        s16 = sa_v[pl.ds(off, _L)]
        d16 = da_v[pl.ds(off, _L)]
        hsv = plsc.load_gather(hs_v, [s16])
        hdv = plsc.load_gather(hd_v, [d16])
        e = hsv + hdv
        e = jnp.where(e > 0, e, 0.2 * e)
        mstab = big_m + hdv
        mstab = jnp.where(mstab > 0, mstab, 0.2 * mstab)
        w = jnp.exp(e - mstab)
        gid = base + off + iota16
        w = jnp.where(gid < _E, w, 0.0)
        return d16, w

    for half in range(2):
        for ch in range(_EPW // _CH):
            hc = c if half == 0 else 1 - c
            base = s * _EPT + hc * _EPW + ch * _CH
            pltpu.sync_copy(src_hbm.at[pl.ds(base, _CH)], sa_v)
            pltpu.sync_copy(dst_hbm.at[pl.ds(base, _CH)], da_v)

            if half == 0:
                def _pa(i, _, base=base, ch=ch):
                    d16, w = _edge_w(base, i)
                    w_v[pl.ds(ch * _CH + i * _L, _L)] = w
                    plsc.addupdate_scatter(ssum_v, [d16 >> 4, d16 & 15], w)
                    return 0
            else:
                def _pa(i, _, base=base):
                    d16, w = _edge_w(base, i)
                    plsc.addupdate_scatter(ssum_v, [d16 >> 4, d16 & 15], w)
                    return 0
            lax.fori_loop(0, _CH // _L, _pa, 0)

    for q in range(_SS // _KB):
        for j in range(_KB // _L):
            id_v[q, pl.ds(j * _L, _L)] = iota16 + (q * _KB + j * _L)
    plsc.subcore_barrier()
    for q in range(_SS // _KB):
        pltpu.sync_copy(ssum_v.at[pl.ds(q * _KB, _KB)],
                        s_sh.at[id_v.at[q]],
                        add=True)
    plsc.subcore_barrier()

    pltpu.sync_copy(s_sh, ssum_v)
    for ch in range(_EPW // _CH):
        pltpu.sync_copy(
            dst_hbm.at[pl.ds(wid * _EPW + ch * _CH, _CH)], da_v)

        def _alpha(i, _, ch=ch):
            d16 = da_v[pl.ds(i * _L, _L)]
            w16 = w_v[pl.ds(ch * _CH + i * _L, _L)]
            sv = plsc.load_gather(ssum_v, [d16 >> 4, d16 & 15])
            w_v[pl.ds(ch * _CH + i * _L, _L)] = w16 / (sv + 1e-16)
            return 0
        lax.fori_loop(0, _CH // _L, _alpha, 0)

    bufs = (rows_v, rows2_v)
    gsems = (gsem0, gsem1)

    for p in range(2):
        for q in range(8):
            ch2 = s * 8 + q
            pltpu.sync_copy(rows_v.at[pl.ds(0, 80)],
                            o_sh.at[pl.ds(ch2 * 80, 80)])
        plsc.subcore_barrier()

        def _g_start(t, b):
            pltpu.async_copy(h2_hbm.at[p].at[srcb_v.at[pl.ds(t * _KB, _KB)]],
                             bufs[b], gsems[b])

        def _g_wait(t, b):
            pltpu.make_async_copy(
                h2_hbm.at[p].at[srcb_v.at[pl.ds(t * _KB, _KB)]],
                bufs[b], gsems[b]).wait()

        def _scale(t, b):
            def _group(g, _):
                a16 = w_v[pl.ds(t * _KB + g * _L, _L)]
                for j in range(_L):
                    r = g * _L + j
                    aj = a16[j]
                    for k in range(_H // _L):
                        bufs[b][r, pl.ds(k * _L, _L)] = (
                            bufs[b][r, pl.ds(k * _L, _L)] * aj)
                return 0
            lax.fori_loop(0, _KB // _L, _group, 0)

        def _pair(i, _):
            t0 = 2 * i
            t1 = 2 * i + 1
            _g_wait(t0, 0)
            _scale(t0, 0)
            d0 = pltpu.async_copy(bufs[0], o_sh.at[dstb_v.at[t0]], ssem,
                                  add=True)
            _g_wait(t1, 1)
            _scale(t1, 1)
            d1 = pltpu.async_copy(bufs[1], o_sh.at[dstb_v.at[t1]], ssem,
                                  add=True)
            d0.wait()

            @pl.when(t0 + 2 < _NB)
            def _():
                _g_start(t0 + 2, 0)
            d1.wait()

            @pl.when(t1 + 2 < _NB)
            def _():
                _g_start(t1 + 2, 1)
            return 0
        _g_start(0, 0)
        _g_start(1, 1)
        lax.fori_loop(0, _NB // 2, _pair, 0)
        plsc.subcore_barrier()

        stripe = _NP // _NS
        pltpu.sync_copy(o_sh.at[pl.ds(s * stripe, stripe)],
                        out_hbm.at[c].at[p].at[pl.ds(s * stripe, stripe)])
        if p == 0:
            plsc.subcore_barrier()

        lax.fori_loop(0, _KB, _zr, 0)


_sc_gat = functools.partial(
    pl.kernel,
    out_type=jax.ShapeDtypeStruct((_NC, _NC, _NP, _H), _f32),
    mesh=plsc.VectorSubcoreMesh(core_axis_name="c", subcore_axis_name="s"),
    compiler_params=pltpu.CompilerParams(needs_layout_passes=False,
                                         use_tc_tiling_on_sc=False),
    scratch_types=[
        pltpu.VMEM((_NP,), _f32),
        pltpu.VMEM((_NP,), _f32),
        pltpu.VMEM((_CH,), jnp.int32),
        pltpu.VMEM((_CH,), jnp.int32),
        pltpu.VMEM((_EPW,), _f32),
        pltpu.VMEM((_EPW,), jnp.int32),
        pltpu.VMEM((_NB, _KB), jnp.int32),
        pltpu.VMEM((_SS // _KB, _KB), jnp.int32),
        pltpu.VMEM((_SS, _L), _f32),
        pltpu.VMEM((_KB, _H), _f32),
        pltpu.VMEM((_KB, _H), _f32),
        pltpu.VMEM_SHARED((_SS, _L), _f32),
        pltpu.VMEM_SHARED((_NP, _H), _f32),
        pltpu.SemaphoreType.DMA,
        pltpu.SemaphoreType.DMA,
        pltpu.SemaphoreType.DMA,
    ],
)(_sc_gat_body)


def kernel(x, edge_index, params):
    src = edge_index[0]
    dst = edge_index[1]
    pad = _EPAD - _E
    src_p = jnp.concatenate([src, jnp.zeros((pad,), jnp.int32)])
    dst_p = jnp.concatenate([dst, jnp.zeros((pad,), jnp.int32)])
    x = jnp.concatenate([x, jnp.zeros((_NP - _N, _C), _f32)])

    convs = params['convs']
    h2, hs, hd = _prep0(x, params['W1'], params['W2'],
                        convs[0]['W'], convs[0]['a_src'], convs[0]['a_dst'])
    for i in range(6):
        o = _sc_gat(h2, hs, hd, src_p, dst_p)
        if i < 5:
            h2, hs, hd = _prepl(o, convs[i]['b'], convs[i + 1]['W'],
                                convs[i + 1]['a_src'], convs[i + 1]['a_dst'])
    return _final(o, convs[5]['b'], params['W3'])[:_N]


# speedup vs baseline: 1.1852x; 1.1852x over previous
import functools

import jax
import jax.numpy as jnp
from jax import lax
from jax.experimental import pallas as pl
from jax.experimental.pallas import tpu as pltpu
from jax.experimental.pallas import tpu_sc as plsc

_N = 10000
_NP = 10240
_E = 320000
_C = 128
_H = _C // 2
_NC = 2
_NS = 16
_NW = _NC * _NS
_L = 16

_EPW = 10240
_EPAD = _EPW * _NW
_EPT = _EPW * _NC
_CH = 2048
_KB = 64
_NB = _EPW // _KB
_SS = _EPW // _L


_BLK = 1024


def _dot(a, b):
    return jnp.dot(a, b, preferred_element_type=jnp.float32)


def _store_h2(h2_ref, hs_ref, hd_ref, h2, asrc, adst):
    h2_ref[0] = h2[:, 0:_H]
    h2_ref[1] = h2[:, _H:_C]
    i = pl.program_id(0)
    hs_ref[pl.ds(i * _BLK, _BLK)] = jnp.sum(h2 * asrc[None, :], axis=1)
    hd_ref[pl.ds(i * _BLK, _BLK)] = jnp.sum(h2 * adst[None, :], axis=1)


def _combine(o_ref, b_ref):
    return jnp.concatenate(
        [o_ref[0, 0] + o_ref[1, 0], o_ref[0, 1] + o_ref[1, 1]],
        axis=1) + b_ref[...][None, :]


def _prep0_body(x_ref, w1_ref, w2_ref, w_ref, asrc_ref, adst_ref,
                h2_ref, hs_ref, hd_ref):
    t = _dot(_dot(x_ref[...], w1_ref[...]), w2_ref[...])
    h2 = _dot(t, w_ref[...])
    _store_h2(h2_ref, hs_ref, hd_ref, h2, asrc_ref[...], adst_ref[...])


def _prepl_body(o_ref, b_ref, w_ref, asrc_ref, adst_ref,
                h2_ref, hs_ref, hd_ref):
    h2 = _dot(_combine(o_ref, b_ref), w_ref[...])
    _store_h2(h2_ref, hs_ref, hd_ref, h2, asrc_ref[...], adst_ref[...])


def _final_body(o_ref, b_ref, w3_ref, out_ref):
    h = jnp.maximum(_combine(o_ref, b_ref), 0.0)
    out_ref[...] = _dot(h, w3_ref[0:_C, :]) + _dot(h, w3_ref[_C:2 * _C, :])


_mat_spec = pl.BlockSpec((_C, _C), lambda i: (0, 0))
_vec_spec = pl.BlockSpec((_C,), lambda i: (0,))
_row_spec = pl.BlockSpec((_BLK, _C), lambda i: (i, 0))
_h2_spec = pl.BlockSpec((_NC, _BLK, _H), lambda i: (0, i, 0))
_par_spec = pl.BlockSpec((_NC, _NC, _BLK, _H), lambda i: (0, 0, i, 0))
_sca_spec = pl.BlockSpec((_NP,), lambda i: (0,))

_f32 = jnp.float32
_h2_shape = jax.ShapeDtypeStruct((_NC, _NP, _H), _f32)
_nv_shape = jax.ShapeDtypeStruct((_NP,), _f32)

_prep0 = pl.pallas_call(
    _prep0_body,
    grid=(_NP // _BLK,),
    in_specs=[_row_spec, _mat_spec, _mat_spec, _mat_spec, _vec_spec, _vec_spec],
    out_specs=[_h2_spec, _sca_spec, _sca_spec],
    out_shape=[_h2_shape, _nv_shape, _nv_shape],
)

_prepl = pl.pallas_call(
    _prepl_body,
    grid=(_NP // _BLK,),
    in_specs=[_par_spec, _vec_spec, _mat_spec, _vec_spec, _vec_spec],
    out_specs=[_h2_spec, _sca_spec, _sca_spec],
    out_shape=[_h2_shape, _nv_shape, _nv_shape],
)

_final = pl.pallas_call(
    _final_body,
    grid=(_NP // _BLK,),
    in_specs=[_par_spec, _vec_spec,
              pl.BlockSpec((2 * _C, _C), lambda i: (0, 0))],
    out_specs=_row_spec,
    out_shape=jax.ShapeDtypeStruct((_NP, _C), _f32),
)


def _sc_gat_body(h2_hbm, hs_hbm, hd_hbm, src_hbm, dst_hbm,
                 out_hbm,
                 hs_v, hd_v, sa_v, da_v, w_v, srcb_v, dstb_v, id_v, ssum_v,
                 rows_v, rows2_v, rows3_v, rows4_v, s_sh, o_sh,
                 gsem0, gsem1, gsem2, gsem3, ssem0, ssem1, ssem2, ssem3):
    c = lax.axis_index("c")
    s = lax.axis_index("s")
    wid = s * _NC + c

    def _vgather(x, idx):
        return lax.gather(
            x, idx[:, None],
            lax.GatherDimensionNumbers(offset_dims=(),
                                       collapsed_slice_dims=(0,),
                                       start_index_map=(0,)),
            (1,), mode=lax.GatherScatterMode.PROMISE_IN_BOUNDS)

    zeros16 = jnp.zeros((_L,), _f32)
    iota16 = lax.iota(jnp.int32, _L)

    pltpu.sync_copy(hs_hbm, hs_v)
    pltpu.sync_copy(hd_hbm, hd_v)
    pltpu.sync_copy(src_hbm.at[pl.ds(wid * _EPW, _EPW)], srcb_v)
    for q in range(_NB):
        pltpu.sync_copy(dst_hbm.at[pl.ds(wid * _EPW + q * _KB, _KB)],
                        dstb_v.at[q])

    def _zs(i, _):
        ssum_v[i] = zeros16
        return 0
    lax.fori_loop(0, _SS, _zs, 0)

    pltpu.sync_copy(ssum_v.at[pl.ds(s * (_SS // _NS), _SS // _NS)],
                    s_sh.at[pl.ds(s * (_SS // _NS), _SS // _NS)])

    def _zr(r, _):
        for k in range(_H // _L):
            rows_v[r, pl.ds(k * _L, _L)] = zeros16
        return 0
    lax.fori_loop(0, _KB, _zr, 0)

    def _mx(i, m):
        return jnp.maximum(m, hs_v[pl.ds(i * _L, _L)])
    m16 = lax.fori_loop(0, _NP // _L, _mx, jnp.full((_L,), -3e38, _f32))
    for k in (8, 4, 2, 1):
        m16 = jnp.maximum(m16, _vgather(m16, iota16 ^ k))
    big_m = m16

    def _edge_w(base, i):
        off = i * _L
        s16 = sa_v[pl.ds(off, _L)]
        d16 = da_v[pl.ds(off, _L)]
        hsv = plsc.load_gather(hs_v, [s16])
        hdv = plsc.load_gather(hd_v, [d16])
        e = hsv + hdv
        e = jnp.where(e > 0, e, 0.2 * e)
        mstab = big_m + hdv
        mstab = jnp.where(mstab > 0, mstab, 0.2 * mstab)
        w = jnp.exp(e - mstab)
        gid = base + off + iota16
        w = jnp.where(gid < _E, w, 0.0)
        return d16, w

    for half in range(2):
        for ch in range(_EPW // _CH):
            hc = c if half == 0 else 1 - c
            base = s * _EPT + hc * _EPW + ch * _CH
            pltpu.sync_copy(src_hbm.at[pl.ds(base, _CH)], sa_v)
            pltpu.sync_copy(dst_hbm.at[pl.ds(base, _CH)], da_v)

            if half == 0:
                def _pa(i, _, base=base, ch=ch):
                    d16, w = _edge_w(base, i)
                    w_v[pl.ds(ch * _CH + i * _L, _L)] = w
                    plsc.addupdate_scatter(ssum_v, [d16 >> 4, d16 & 15], w)
                    return 0
            else:
                def _pa(i, _, base=base):
                    d16, w = _edge_w(base, i)
                    plsc.addupdate_scatter(ssum_v, [d16 >> 4, d16 & 15], w)
                    return 0
            lax.fori_loop(0, _CH // _L, _pa, 0)

    for q in range(_SS // _KB):
        for j in range(_KB // _L):
            id_v[q, pl.ds(j * _L, _L)] = iota16 + (q * _KB + j * _L)
    plsc.subcore_barrier()
    for q in range(_SS // _KB):
        pltpu.sync_copy(ssum_v.at[pl.ds(q * _KB, _KB)],
                        s_sh.at[id_v.at[q]],
                        add=True)
    plsc.subcore_barrier()

    pltpu.sync_copy(s_sh, ssum_v)
    for ch in range(_EPW // _CH):
        pltpu.sync_copy(
            dst_hbm.at[pl.ds(wid * _EPW + ch * _CH, _CH)], da_v)

        def _alpha(i, _, ch=ch):
            d16 = da_v[pl.ds(i * _L, _L)]
            w16 = w_v[pl.ds(ch * _CH + i * _L, _L)]
            sv = plsc.load_gather(ssum_v, [d16 >> 4, d16 & 15])
            w_v[pl.ds(ch * _CH + i * _L, _L)] = w16 / (sv + 1e-16)
            return 0
        lax.fori_loop(0, _CH // _L, _alpha, 0)

    bufs = (rows_v, rows2_v, rows3_v, rows4_v)
    gsems = (gsem0, gsem1, gsem2, gsem3)
    ssems = (ssem0, ssem1, ssem2, ssem3)

    for p in range(2):
        for q in range(10):
            ch2 = s * 10 + q
            pltpu.sync_copy(rows_v.at[pl.ds(0, _KB)],
                            o_sh.at[pl.ds(ch2 * _KB, _KB)])
        plsc.subcore_barrier()

        def _g_start(t, b):
            pltpu.async_copy(h2_hbm.at[p].at[srcb_v.at[pl.ds(t * _KB, _KB)]],
                             bufs[b], gsems[b])

        def _g_wait(t, b):
            pltpu.make_async_copy(
                h2_hbm.at[p].at[srcb_v.at[pl.ds(t * _KB, _KB)]],
                bufs[b], gsems[b]).wait()

        def _scale(t, b):
            def _group(g, _):
                a16 = w_v[pl.ds(t * _KB + g * _L, _L)]
                for j in range(_L):
                    r = g * _L + j
                    aj = a16[j]
                    for k in range(_H // _L):
                        bufs[b][r, pl.ds(k * _L, _L)] = (
                            bufs[b][r, pl.ds(k * _L, _L)] * aj)
                return 0
            lax.fori_loop(0, _KB // _L, _group, 0)

        def _s_start(t, b):
            pltpu.async_copy(bufs[b], o_sh.at[dstb_v.at[t]], ssems[b],
                             add=True)

        def _s_wait(t, b):
            pltpu.make_async_copy(bufs[b], o_sh.at[dstb_v.at[t]],
                                  ssems[b]).wait()

        def _step(t, b):
            _g_wait(t, b)
            _scale(t, b)
            _s_start(t, b)

            @pl.when(t >= 2)
            def _():
                _s_wait(t - 2, (b - 2) % 4)

            @pl.when(t + 2 < _NB)
            def _():
                _g_start(t + 2, (b + 2) % 4)

        def _quad(i, _):
            for u in range(4):
                _step(4 * i + u, u)
            return 0
        _g_start(0, 0)
        _g_start(1, 1)
        lax.fori_loop(0, _NB // 4, _quad, 0)
        _s_wait(_NB - 2, 2)
        _s_wait(_NB - 1, 3)
        plsc.subcore_barrier()

        stripe = _NP // _NS
        pltpu.sync_copy(o_sh.at[pl.ds(s * stripe, stripe)],
                        out_hbm.at[c].at[p].at[pl.ds(s * stripe, stripe)])
        if p == 0:
            plsc.subcore_barrier()

        lax.fori_loop(0, _KB, _zr, 0)


_sc_gat = functools.partial(
    pl.kernel,
    out_type=jax.ShapeDtypeStruct((_NC, _NC, _NP, _H), _f32),
    mesh=plsc.VectorSubcoreMesh(core_axis_name="c", subcore_axis_name="s"),
    compiler_params=pltpu.CompilerParams(needs_layout_passes=False,
                                         use_tc_tiling_on_sc=False),
    scratch_types=[
        pltpu.VMEM((_NP,), _f32),
        pltpu.VMEM((_NP,), _f32),
        pltpu.VMEM((_CH,), jnp.int32),
        pltpu.VMEM((_CH,), jnp.int32),
        pltpu.VMEM((_EPW,), _f32),
        pltpu.VMEM((_EPW,), jnp.int32),
        pltpu.VMEM((_NB, _KB), jnp.int32),
        pltpu.VMEM((_SS // _KB, _KB), jnp.int32),
        pltpu.VMEM((_SS, _L), _f32),
        pltpu.VMEM((_KB, _H), _f32),
        pltpu.VMEM((_KB, _H), _f32),
        pltpu.VMEM((_KB, _H), _f32),
        pltpu.VMEM((_KB, _H), _f32),
        pltpu.VMEM_SHARED((_SS, _L), _f32),
        pltpu.VMEM_SHARED((_NP, _H), _f32),
    ] + [pltpu.SemaphoreType.DMA] * 8,
)(_sc_gat_body)


def kernel(x, edge_index, params):
    src = edge_index[0]
    dst = edge_index[1]
    pad = _EPAD - _E
    src_p = jnp.concatenate([src, jnp.zeros((pad,), jnp.int32)])
    dst_p = jnp.concatenate([dst, jnp.zeros((pad,), jnp.int32)])
    x = jnp.concatenate([x, jnp.zeros((_NP - _N, _C), _f32)])

    convs = params['convs']
    h2, hs, hd = _prep0(x, params['W1'], params['W2'],
                        convs[0]['W'], convs[0]['a_src'], convs[0]['a_dst'])
    for i in range(6):
        o = _sc_gat(h2, hs, hd, src_p, dst_p)
        if i < 5:
            h2, hs, hd = _prepl(o, convs[i]['b'], convs[i + 1]['W'],
                                convs[i + 1]['a_src'], convs[i + 1]['a_dst'])
    return _final(o, convs[5]['b'], params['W3'])[:_N]


# speedup vs baseline: 1.3885x; 1.1715x over previous
import functools

import jax
import jax.numpy as jnp
from jax import lax
from jax.experimental import pallas as pl
from jax.experimental.pallas import tpu as pltpu
from jax.experimental.pallas import tpu_sc as plsc

_N = 10000
_NP = 10240
_E = 320000
_C = 128
_H = _C // 2
_NC = 2
_NS = 16
_NW = _NC * _NS
_L = 16

_EPW = 10240
_EPAD = _EPW * _NW
_EPT = _EPW * _NC
_CH = 2048
_KB = 64
_NB = _EPW // _KB
_SS = _EPW // _L


_BLK = 1024


def _dot(a, b):
    return jnp.dot(a, b, preferred_element_type=jnp.float32)


def _store_h2(h2_ref, hs_ref, hd_ref, h2, asrc, adst):
    h2_ref[0] = h2[:, 0:_H]
    h2_ref[1] = h2[:, _H:_C]
    i = pl.program_id(0)
    hs_ref[pl.ds(i * _BLK, _BLK)] = jnp.sum(h2 * asrc[None, :], axis=1)
    hd_ref[pl.ds(i * _BLK, _BLK)] = jnp.sum(h2 * adst[None, :], axis=1)


def _combine(o_ref, b_ref):
    return jnp.concatenate(
        [o_ref[0, 0] + o_ref[1, 0], o_ref[0, 1] + o_ref[1, 1]],
        axis=1) + b_ref[...][None, :]


def _prep0_body(x_ref, w1_ref, w2_ref, w_ref, asrc_ref, adst_ref,
                h2_ref, hs_ref, hd_ref):
    t = _dot(_dot(x_ref[...], w1_ref[...]), w2_ref[...])
    h2 = _dot(t, w_ref[...])
    _store_h2(h2_ref, hs_ref, hd_ref, h2, asrc_ref[...], adst_ref[...])


def _prepl_body(o_ref, b_ref, w_ref, asrc_ref, adst_ref,
                h2_ref, hs_ref, hd_ref):
    h2 = _dot(_combine(o_ref, b_ref), w_ref[...])
    _store_h2(h2_ref, hs_ref, hd_ref, h2, asrc_ref[...], adst_ref[...])


def _final_body(o_ref, b_ref, w3_ref, out_ref):
    h = jnp.maximum(_combine(o_ref, b_ref), 0.0)
    out_ref[...] = _dot(h, w3_ref[0:_C, :]) + _dot(h, w3_ref[_C:2 * _C, :])


_mat_spec = pl.BlockSpec((_C, _C), lambda i: (0, 0))
_vec_spec = pl.BlockSpec((_C,), lambda i: (0,))
_row_spec = pl.BlockSpec((_BLK, _C), lambda i: (i, 0))
_h2_spec = pl.BlockSpec((_NC, _BLK, _H), lambda i: (0, i, 0))
_par_spec = pl.BlockSpec((_NC, _NC, _BLK, _H), lambda i: (0, 0, i, 0))
_sca_spec = pl.BlockSpec((_NP,), lambda i: (0,))

_f32 = jnp.float32
_h2_shape = jax.ShapeDtypeStruct((_NC, _NP, _H), _f32)
_nv_shape = jax.ShapeDtypeStruct((_NP,), _f32)

_prep0 = pl.pallas_call(
    _prep0_body,
    grid=(_NP // _BLK,),
    in_specs=[_row_spec, _mat_spec, _mat_spec, _mat_spec, _vec_spec, _vec_spec],
    out_specs=[_h2_spec, _sca_spec, _sca_spec],
    out_shape=[_h2_shape, _nv_shape, _nv_shape],
)

_prepl = pl.pallas_call(
    _prepl_body,
    grid=(_NP // _BLK,),
    in_specs=[_par_spec, _vec_spec, _mat_spec, _vec_spec, _vec_spec],
    out_specs=[_h2_spec, _sca_spec, _sca_spec],
    out_shape=[_h2_shape, _nv_shape, _nv_shape],
)

_final = pl.pallas_call(
    _final_body,
    grid=(_NP // _BLK,),
    in_specs=[_par_spec, _vec_spec,
              pl.BlockSpec((2 * _C, _C), lambda i: (0, 0))],
    out_specs=_row_spec,
    out_shape=jax.ShapeDtypeStruct((_NP, _C), _f32),
)


def _sc_gat_body(h2_hbm, hs_hbm, hd_hbm, src3_hbm, dst3_hbm,
                 out_hbm,
                 hs_v, hd_v, sa_v, da_v, w_v, srcb_v, dstb_v, id_v, ssum_v,
                 rows_v, rows2_v, rows3_v, rows4_v, s_sh, o_sh,
                 gsem0, gsem1, gsem2, gsem3, ssem0, ssem1, ssem2, ssem3):
    c = lax.axis_index("c")
    s = lax.axis_index("s")
    wid = s * _NC + c

    def _vgather(x, idx):
        return lax.gather(
            x, idx[:, None],
            lax.GatherDimensionNumbers(offset_dims=(),
                                       collapsed_slice_dims=(0,),
                                       start_index_map=(0,)),
            (1,), mode=lax.GatherScatterMode.PROMISE_IN_BOUNDS)

    zeros16 = jnp.zeros((_L,), _f32)
    iota16 = lax.iota(jnp.int32, _L)

    pltpu.sync_copy(hs_hbm, hs_v)
    pltpu.sync_copy(hd_hbm, hd_v)
    pltpu.sync_copy(src3_hbm.at[wid], srcb_v)
    pltpu.sync_copy(dst3_hbm.at[wid], dstb_v)

    def _zs(i, _):
        ssum_v[i] = zeros16
        return 0
    lax.fori_loop(0, _SS, _zs, 0)

    pltpu.sync_copy(ssum_v.at[pl.ds(s * (_SS // _NS), _SS // _NS)],
                    s_sh.at[pl.ds(s * (_SS // _NS), _SS // _NS)])

    def _zr(r, _):
        for k in range(_H // _L):
            rows_v[r, pl.ds(k * _L, _L)] = zeros16
        return 0
    lax.fori_loop(0, _KB, _zr, 0)

    def _mx(i, m):
        return jnp.maximum(m, hs_v[pl.ds(i * _L, _L)])
    m16 = lax.fori_loop(0, _NP // _L, _mx, jnp.full((_L,), -3e38, _f32))
    for k in (8, 4, 2, 1):
        m16 = jnp.maximum(m16, _vgather(m16, iota16 ^ k))
    big_m = m16

    _RPC = _CH // _KB

    def _edge_w(base, r, j):
        s16 = sa_v[r, pl.ds(j * _L, _L)]
        d16 = da_v[r, pl.ds(j * _L, _L)]
        hsv = plsc.load_gather(hs_v, [s16])
        hdv = plsc.load_gather(hd_v, [d16])
        e = hsv + hdv
        e = jnp.where(e > 0, e, 0.2 * e)
        mstab = big_m + hdv
        mstab = jnp.where(mstab > 0, mstab, 0.2 * mstab)
        w = jnp.exp(e - mstab)
        gid = base + r * _KB + j * _L + iota16
        w = jnp.where(gid < _E, w, 0.0)
        return d16, w

    for half in range(2):
        for ch in range(_EPW // _CH):
            hc = c if half == 0 else 1 - c
            wa = s * _NC + hc
            base = wa * _EPW + ch * _CH
            pltpu.sync_copy(src3_hbm.at[wa].at[pl.ds(ch * _RPC, _RPC)], sa_v)
            pltpu.sync_copy(dst3_hbm.at[wa].at[pl.ds(ch * _RPC, _RPC)], da_v)

            if half == 0:
                def _pa(r, _, base=base, ch=ch):
                    for j in range(_KB // _L):
                        d16, w = _edge_w(base, r, j)
                        w_v[pl.ds(ch * _CH + r * _KB + j * _L, _L)] = w
                        plsc.addupdate_scatter(ssum_v,
                                               [d16 >> 4, d16 & 15], w)
                    return 0
            else:
                def _pa(r, _, base=base):
                    for j in range(_KB // _L):
                        d16, w = _edge_w(base, r, j)
                        plsc.addupdate_scatter(ssum_v,
                                               [d16 >> 4, d16 & 15], w)
                    return 0
            lax.fori_loop(0, _RPC, _pa, 0)

    for q in range(_SS // _KB):
        for j in range(_KB // _L):
            id_v[q, pl.ds(j * _L, _L)] = iota16 + (q * _KB + j * _L)
    plsc.subcore_barrier()
    for q in range(_SS // _KB):
        pltpu.sync_copy(ssum_v.at[pl.ds(q * _KB, _KB)],
                        s_sh.at[id_v.at[q]],
                        add=True)
    plsc.subcore_barrier()

    pltpu.sync_copy(s_sh, ssum_v)
    for ch in range(_EPW // _CH):
        pltpu.sync_copy(dst3_hbm.at[wid].at[pl.ds(ch * _RPC, _RPC)], da_v)

        def _alpha(r, _, ch=ch):
            for j in range(_KB // _L):
                d16 = da_v[r, pl.ds(j * _L, _L)]
                pos = ch * _CH + r * _KB + j * _L
                w16 = w_v[pl.ds(pos, _L)]
                sv = plsc.load_gather(ssum_v, [d16 >> 4, d16 & 15])
                w_v[pl.ds(pos, _L)] = w16 / (sv + 1e-16)
            return 0
        lax.fori_loop(0, _RPC, _alpha, 0)

    bufs = (rows_v, rows2_v, rows3_v, rows4_v)
    gsems = (gsem0, gsem1, gsem2, gsem3)
    ssems = (ssem0, ssem1, ssem2, ssem3)

    for p in range(2):
        for q in range(10):
            ch2 = s * 10 + q
            pltpu.sync_copy(rows_v.at[pl.ds(0, _KB)],
                            o_sh.at[pl.ds(ch2 * _KB, _KB)])
        plsc.subcore_barrier()

        def _g_start(t, b):
            pltpu.async_copy(h2_hbm.at[p].at[srcb_v.at[t]],
                             bufs[b], gsems[b])

        def _g_wait(t, b):
            pltpu.make_async_copy(
                h2_hbm.at[p].at[srcb_v.at[t]],
                bufs[b], gsems[b]).wait()

        def _scale(t, b):
            def _group(g, _):
                a16 = w_v[pl.ds(t * _KB + g * _L, _L)]
                for j in range(_L):
                    r = g * _L + j
                    aj = a16[j]
                    for k in range(_H // _L):
                        bufs[b][r, pl.ds(k * _L, _L)] = (
                            bufs[b][r, pl.ds(k * _L, _L)] * aj)
                return 0
            lax.fori_loop(0, _KB // _L, _group, 0)

        def _s_start(t, b):
            pltpu.async_copy(bufs[b], o_sh.at[dstb_v.at[t]], ssems[b],
                             add=True)

        def _s_wait(t, b):
            pltpu.make_async_copy(bufs[b], o_sh.at[dstb_v.at[t]],
                                  ssems[b]).wait()

        def _step(t, b):
            _g_wait(t, b)
            _scale(t, b)
            _s_start(t, b)

            @pl.when(t >= 2)
            def _():
                _s_wait(t - 2, (b - 2) % 4)

            @pl.when(t + 2 < _NB)
            def _():
                _g_start(t + 2, (b + 2) % 4)

        def _quad(i, _):
            for u in range(4):
                _step(4 * i + u, u)
            return 0
        _g_start(0, 0)
        _g_start(1, 1)
        lax.fori_loop(0, _NB // 4, _quad, 0)
        _s_wait(_NB - 2, 2)
        _s_wait(_NB - 1, 3)
        plsc.subcore_barrier()

        stripe = _NP // _NS
        pltpu.sync_copy(o_sh.at[pl.ds(s * stripe, stripe)],
                        out_hbm.at[c].at[p].at[pl.ds(s * stripe, stripe)])
        if p == 0:
            plsc.subcore_barrier()

        lax.fori_loop(0, _KB, _zr, 0)


_sc_gat = functools.partial(
    pl.kernel,
    out_type=jax.ShapeDtypeStruct((_NC, _NC, _NP, _H), _f32),
    mesh=plsc.VectorSubcoreMesh(core_axis_name="c", subcore_axis_name="s"),
    compiler_params=pltpu.CompilerParams(needs_layout_passes=False,
                                         use_tc_tiling_on_sc=False),
    scratch_types=[
        pltpu.VMEM((_NP,), _f32),
        pltpu.VMEM((_NP,), _f32),
        pltpu.VMEM((_CH // _KB, _KB), jnp.int32),
        pltpu.VMEM((_CH // _KB, _KB), jnp.int32),
        pltpu.VMEM((_EPW,), _f32),
        pltpu.VMEM((_NB, _KB), jnp.int32),
        pltpu.VMEM((_NB, _KB), jnp.int32),
        pltpu.VMEM((_SS // _KB, _KB), jnp.int32),
        pltpu.VMEM((_SS, _L), _f32),
        pltpu.VMEM((_KB, _H), _f32),
        pltpu.VMEM((_KB, _H), _f32),
        pltpu.VMEM((_KB, _H), _f32),
        pltpu.VMEM((_KB, _H), _f32),
        pltpu.VMEM_SHARED((_SS, _L), _f32),
        pltpu.VMEM_SHARED((_NP, _H), _f32),
    ] + [pltpu.SemaphoreType.DMA] * 8,
)(_sc_gat_body)


def kernel(x, edge_index, params):
    src = edge_index[0]
    dst = edge_index[1]
    pad = _EPAD - _E
    src_p = jnp.concatenate([src, jnp.zeros((pad,), jnp.int32)])
    dst_p = jnp.concatenate([dst, jnp.zeros((pad,), jnp.int32)])
    x = jnp.concatenate([x, jnp.zeros((_NP - _N, _C), _f32)])
    src3 = src_p.reshape(_NW, _NB, _KB)
    dst3 = dst_p.reshape(_NW, _NB, _KB)

    convs = params['convs']
    h2, hs, hd = _prep0(x, params['W1'], params['W2'],
                        convs[0]['W'], convs[0]['a_src'], convs[0]['a_dst'])
    for i in range(6):
        o = _sc_gat(h2, hs, hd, src3, dst3)
        if i < 5:
            h2, hs, hd = _prepl(o, convs[i]['b'], convs[i + 1]['W'],
                                convs[i + 1]['a_src'], convs[i + 1]['a_dst'])
    return _final(o, convs[5]['b'], params['W3'])[:_N]


# speedup vs baseline: 1.4124x; 1.0172x over previous
import functools

import jax
import jax.numpy as jnp
from jax import lax
from jax.experimental import pallas as pl
from jax.experimental.pallas import tpu as pltpu
from jax.experimental.pallas import tpu_sc as plsc

_N = 10000
_NP = 10240
_E = 320000
_C = 128
_H = _C // 2
_NC = 2
_NS = 16
_NW = _NC * _NS
_L = 16

_EPW = 10240
_EPAD = _EPW * _NW
_EPT = _EPW * _NC
_CH = 2048
_KB = 64
_NB = _EPW // _KB
_SS = _EPW // _L


_BLK = 1024


def _dot(a, b):
    return jnp.dot(a, b, preferred_element_type=jnp.float32)


def _store_h2(h2_ref, hs_ref, hd_ref, h2, asrc, adst):
    h2_ref[0] = h2[:, 0:_H]
    h2_ref[1] = h2[:, _H:_C]
    i = pl.program_id(0)
    hs_ref[pl.ds(i * _BLK, _BLK)] = jnp.sum(h2 * asrc[None, :], axis=1)
    hd_ref[pl.ds(i * _BLK, _BLK)] = jnp.sum(h2 * adst[None, :], axis=1)


def _combine(o_ref, b_ref):
    return jnp.concatenate(
        [o_ref[0, 0] + o_ref[1, 0], o_ref[0, 1] + o_ref[1, 1]],
        axis=1) + b_ref[...][None, :]


def _prep0_body(x_ref, w1_ref, w2_ref, w_ref, asrc_ref, adst_ref,
                h2_ref, hs_ref, hd_ref):
    t = _dot(_dot(x_ref[...], w1_ref[...]), w2_ref[...])
    h2 = _dot(t, w_ref[...])
    _store_h2(h2_ref, hs_ref, hd_ref, h2, asrc_ref[...], adst_ref[...])


def _prepl_body(o_ref, b_ref, w_ref, asrc_ref, adst_ref,
                h2_ref, hs_ref, hd_ref):
    h2 = _dot(_combine(o_ref, b_ref), w_ref[...])
    _store_h2(h2_ref, hs_ref, hd_ref, h2, asrc_ref[...], adst_ref[...])


def _final_body(o_ref, b_ref, w3_ref, out_ref):
    h = jnp.maximum(_combine(o_ref, b_ref), 0.0)
    out_ref[...] = _dot(h, w3_ref[0:_C, :]) + _dot(h, w3_ref[_C:2 * _C, :])


_mat_spec = pl.BlockSpec((_C, _C), lambda i: (0, 0))
_vec_spec = pl.BlockSpec((_C,), lambda i: (0,))
_row_spec = pl.BlockSpec((_BLK, _C), lambda i: (i, 0))
_h2_spec = pl.BlockSpec((_NC, _BLK, _H), lambda i: (0, i, 0))
_par_spec = pl.BlockSpec((_NC, _NC, _BLK, _H), lambda i: (0, 0, i, 0))
_sca_spec = pl.BlockSpec((_NP,), lambda i: (0,))

_f32 = jnp.float32
_h2_shape = jax.ShapeDtypeStruct((_NC, _NP, _H), _f32)
_nv_shape = jax.ShapeDtypeStruct((_NP,), _f32)

_prep0 = pl.pallas_call(
    _prep0_body,
    grid=(_NP // _BLK,),
    in_specs=[_row_spec, _mat_spec, _mat_spec, _mat_spec, _vec_spec, _vec_spec],
    out_specs=[_h2_spec, _sca_spec, _sca_spec],
    out_shape=[_h2_shape, _nv_shape, _nv_shape],
)

_prepl = pl.pallas_call(
    _prepl_body,
    grid=(_NP // _BLK,),
    in_specs=[_par_spec, _vec_spec, _mat_spec, _vec_spec, _vec_spec],
    out_specs=[_h2_spec, _sca_spec, _sca_spec],
    out_shape=[_h2_shape, _nv_shape, _nv_shape],
)

_final = pl.pallas_call(
    _final_body,
    grid=(_NP // _BLK,),
    in_specs=[_par_spec, _vec_spec,
              pl.BlockSpec((2 * _C, _C), lambda i: (0, 0))],
    out_specs=_row_spec,
    out_shape=jax.ShapeDtypeStruct((_NP, _C), _f32),
)


def _sc_gat_body(h2_hbm, hs_hbm, hd_hbm, src3_hbm, dst3_hbm,
                 out_hbm,
                 hs_v, hd_v, sa_v, da_v, w_v, srcb_v, dstb_v, id_v, ssum_v,
                 rows_v, rows2_v, rows3_v, rows4_v, s_sh, o_sh,
                 gsem0, gsem1, gsem2, gsem3, ssem0, ssem1, ssem2, ssem3):
    c = lax.axis_index("c")
    s = lax.axis_index("s")
    wid = s * _NC + c

    def _vgather(x, idx):
        return lax.gather(
            x, idx[:, None],
            lax.GatherDimensionNumbers(offset_dims=(),
                                       collapsed_slice_dims=(0,),
                                       start_index_map=(0,)),
            (1,), mode=lax.GatherScatterMode.PROMISE_IN_BOUNDS)

    zeros16 = jnp.zeros((_L,), _f32)
    iota16 = lax.iota(jnp.int32, _L)

    pltpu.sync_copy(hs_hbm, hs_v)
    pltpu.sync_copy(hd_hbm, hd_v)
    pltpu.sync_copy(src3_hbm.at[wid], srcb_v)
    pltpu.sync_copy(dst3_hbm.at[wid], dstb_v)

    def _zs(i, _):
        ssum_v[i] = zeros16
        return 0
    lax.fori_loop(0, _SS, _zs, 0)

    pltpu.sync_copy(ssum_v.at[pl.ds(s * (_SS // _NS), _SS // _NS)],
                    s_sh.at[pl.ds(s * (_SS // _NS), _SS // _NS)])

    def _zr(r, _):
        for k in range(_H // _L):
            rows_v[r, pl.ds(k * _L, _L)] = zeros16
        return 0
    lax.fori_loop(0, _KB, _zr, 0)

    def _mx(i, m):
        return jnp.maximum(m, hs_v[pl.ds(i * _L, _L)])
    m16 = lax.fori_loop(0, _NP // _L, _mx, jnp.full((_L,), -3e38, _f32))
    for k in (8, 4, 2, 1):
        m16 = jnp.maximum(m16, _vgather(m16, iota16 ^ k))
    big_m = m16

    _RPC = _CH // _KB

    def _edge_w(sref, dref, base, r, j):
        s16 = sref[r, pl.ds(j * _L, _L)]
        d16 = dref[r, pl.ds(j * _L, _L)]
        hsv = plsc.load_gather(hs_v, [s16])
        hdv = plsc.load_gather(hd_v, [d16])
        e = hsv + hdv
        e = jnp.where(e > 0, e, 0.2 * e)
        mstab = big_m + hdv
        mstab = jnp.where(mstab > 0, mstab, 0.2 * mstab)
        w = jnp.exp(e - mstab)
        gid = base + r * _KB + j * _L + iota16
        w = jnp.where(gid < _E, w, 0.0)
        return d16, w

    def _pa_own(r, _):
        for j in range(_KB // _L):
            d16, w = _edge_w(srcb_v, dstb_v, wid * _EPW, r, j)
            w_v[pl.ds(r * _KB + j * _L, _L)] = w
            plsc.addupdate_scatter(ssum_v, [d16 >> 4, d16 & 15], w)
        return 0
    lax.fori_loop(0, _NB, _pa_own, 0)

    wt = s * _NC + (1 - c)
    for ch in range(_EPW // _CH):
        base = wt * _EPW + ch * _CH
        pltpu.sync_copy(src3_hbm.at[wt].at[pl.ds(ch * _RPC, _RPC)], sa_v)
        pltpu.sync_copy(dst3_hbm.at[wt].at[pl.ds(ch * _RPC, _RPC)], da_v)

        def _pa(r, _, base=base):
            for j in range(_KB // _L):
                d16, w = _edge_w(sa_v, da_v, base, r, j)
                plsc.addupdate_scatter(ssum_v, [d16 >> 4, d16 & 15], w)
            return 0
        lax.fori_loop(0, _RPC, _pa, 0)

    for q in range(_SS // _KB):
        for j in range(_KB // _L):
            id_v[q, pl.ds(j * _L, _L)] = iota16 + (q * _KB + j * _L)
    plsc.subcore_barrier()
    for q in range(_SS // _KB):
        pltpu.sync_copy(ssum_v.at[pl.ds(q * _KB, _KB)],
                        s_sh.at[id_v.at[q]],
                        add=True)
    plsc.subcore_barrier()

    pltpu.sync_copy(s_sh, ssum_v)
    def _alpha(r, _):
        for j in range(_KB // _L):
            d16 = dstb_v[r, pl.ds(j * _L, _L)]
            pos = r * _KB + j * _L
            w16 = w_v[pl.ds(pos, _L)]
            sv = plsc.load_gather(ssum_v, [d16 >> 4, d16 & 15])
            w_v[pl.ds(pos, _L)] = w16 / (sv + 1e-16)
        return 0
    lax.fori_loop(0, _NB, _alpha, 0)

    bufs = (rows_v, rows2_v, rows3_v, rows4_v)
    gsems = (gsem0, gsem1, gsem2, gsem3)
    ssems = (ssem0, ssem1, ssem2, ssem3)

    for p in range(2):
        for q in range(10):
            ch2 = s * 10 + q
            pltpu.sync_copy(rows_v.at[pl.ds(0, _KB)],
                            o_sh.at[pl.ds(ch2 * _KB, _KB)])
        plsc.subcore_barrier()

        def _g_start(t, b):
            pltpu.async_copy(h2_hbm.at[p].at[srcb_v.at[t]],
                             bufs[b], gsems[b])

        def _g_wait(t, b):
            pltpu.make_async_copy(
                h2_hbm.at[p].at[srcb_v.at[t]],
                bufs[b], gsems[b]).wait()

        def _scale(t, b):
            def _group(g, _):
                a16 = w_v[pl.ds(t * _KB + g * _L, _L)]
                for j in range(_L):
                    r = g * _L + j
                    aj = a16[j]
                    for k in range(_H // _L):
                        bufs[b][r, pl.ds(k * _L, _L)] = (
                            bufs[b][r, pl.ds(k * _L, _L)] * aj)
                return 0
            lax.fori_loop(0, _KB // _L, _group, 0)

        def _s_start(t, b):
            pltpu.async_copy(bufs[b], o_sh.at[dstb_v.at[t]], ssems[b],
                             add=True)

        def _s_wait(t, b):
            pltpu.make_async_copy(bufs[b], o_sh.at[dstb_v.at[t]],
                                  ssems[b]).wait()

        def _step(t, b):
            _g_wait(t, b)
            _scale(t, b)
            _s_start(t, b)

            @pl.when(t >= 2)
            def _():
                _s_wait(t - 2, (b - 2) % 4)

            @pl.when(t + 2 < _NB)
            def _():
                _g_start(t + 2, (b + 2) % 4)

        def _quad(i, _):
            for u in range(4):
                _step(4 * i + u, u)
            return 0
        _g_start(0, 0)
        _g_start(1, 1)
        lax.fori_loop(0, _NB // 4, _quad, 0)
        _s_wait(_NB - 2, 2)
        _s_wait(_NB - 1, 3)
        plsc.subcore_barrier()

        stripe = _NP // _NS
        pltpu.sync_copy(o_sh.at[pl.ds(s * stripe, stripe)],
                        out_hbm.at[c].at[p].at[pl.ds(s * stripe, stripe)])
        if p == 0:
            plsc.subcore_barrier()

        lax.fori_loop(0, _KB, _zr, 0)


_sc_gat = functools.partial(
    pl.kernel,
    out_type=jax.ShapeDtypeStruct((_NC, _NC, _NP, _H), _f32),
    mesh=plsc.VectorSubcoreMesh(core_axis_name="c", subcore_axis_name="s"),
    compiler_params=pltpu.CompilerParams(needs_layout_passes=False,
                                         use_tc_tiling_on_sc=False),
    scratch_types=[
        pltpu.VMEM((_NP,), _f32),
        pltpu.VMEM((_NP,), _f32),
        pltpu.VMEM((_CH // _KB, _KB), jnp.int32),
        pltpu.VMEM((_CH // _KB, _KB), jnp.int32),
        pltpu.VMEM((_EPW,), _f32),
        pltpu.VMEM((_NB, _KB), jnp.int32),
        pltpu.VMEM((_NB, _KB), jnp.int32),
        pltpu.VMEM((_SS // _KB, _KB), jnp.int32),
        pltpu.VMEM((_SS, _L), _f32),
        pltpu.VMEM((_KB, _H), _f32),
        pltpu.VMEM((_KB, _H), _f32),
        pltpu.VMEM((_KB, _H), _f32),
        pltpu.VMEM((_KB, _H), _f32),
        pltpu.VMEM_SHARED((_SS, _L), _f32),
        pltpu.VMEM_SHARED((_NP, _H), _f32),
    ] + [pltpu.SemaphoreType.DMA] * 8,
)(_sc_gat_body)


def kernel(x, edge_index, params):
    src = edge_index[0]
    dst = edge_index[1]
    pad = _EPAD - _E
    src_p = jnp.concatenate([src, jnp.zeros((pad,), jnp.int32)])
    dst_p = jnp.concatenate([dst, jnp.zeros((pad,), jnp.int32)])
    x = jnp.concatenate([x, jnp.zeros((_NP - _N, _C), _f32)])
    src3 = src_p.reshape(_NW, _NB, _KB)
    dst3 = dst_p.reshape(_NW, _NB, _KB)

    convs = params['convs']
    h2, hs, hd = _prep0(x, params['W1'], params['W2'],
                        convs[0]['W'], convs[0]['a_src'], convs[0]['a_dst'])
    for i in range(6):
        o = _sc_gat(h2, hs, hd, src3, dst3)
        if i < 5:
            h2, hs, hd = _prepl(o, convs[i]['b'], convs[i + 1]['W'],
                                convs[i + 1]['a_src'], convs[i + 1]['a_dst'])
    return _final(o, convs[5]['b'], params['W3'])[:_N]
